# Initial kernel scaffold; baseline (speedup 1.0000x reference)
import jax
import jax.numpy as jnp
from jax.experimental import pallas as pl


def kernel(node_inp, node_type, edge_index, edge_type, Wk, bk, Wq, bq, Wv, bv, Wa, ba, relation_pri, relation_att, relation_msg, skip, ln_g, ln_b):
    raise NotImplementedError("write your pallas kernel here")


# speedup vs baseline: 2.0731x; 2.0731x over previous
import jax
import jax.numpy as jnp
from jax import lax
from jax.experimental import pallas as pl
from jax.experimental.pallas import tpu as pltpu
from jax.experimental.pallas import tpu_sc as plsc

N = 10000
E = 160000
IN_DIM = 256
OUT_DIM = 256
NUM_TYPES = 4
NUM_RELATIONS = 4
N_HEADS = 8
D_K = 32
SQRT_DK = float(32) ** 0.5

NPAD = 10240
BN = 512
NB = NPAD // BN
NC = 2
NS = 16
CHUNK = 64
NCH = 160
EPT = NCH * CHUNK
EPAD = NS * EPT
HH = 128
HN = NPAD // 2
HN_ACC = HN + 128
ZPT = HN_ACC // NS
DPT = HN // NS
DPKR = 648

def _take(v, idx):
    return v.at[idx].get(mode="promise_in_bounds")


def _stage1_body(x_ref, m_ref, wk_ref, bk_ref, wq_ref, bq_ref, wv_ref,
                 bv_ref, bda_ref, bdm_ref, qh_ref, kr_ref, vm_ref):
    x = x_ref[...]
    m = m_ref[...]

    def typed(w_ref, b_ref):
        acc = jnp.zeros((BN, OUT_DIM), jnp.float32)
        for t in range(NUM_TYPES):
            y = jnp.dot(x, w_ref[t], preferred_element_type=jnp.float32)
            y = y + b_ref[t][None, :]
            acc = acc + y * m[:, t][:, None]
        return acc

    k = typed(wk_ref, bk_ref)
    q = typed(wq_ref, bq_ref)
    v = typed(wv_ref, bv_ref)
    qh_ref[0] = q[:, :HH]
    qh_ref[1] = q[:, HH:]
    for r in range(NUM_RELATIONS):
        krr = jnp.dot(k, bda_ref[r], preferred_element_type=jnp.float32)
        vmr = jnp.dot(v, bdm_ref[r], preferred_element_type=jnp.float32)
        kr_ref[0, r] = krr[:, :HH]
        kr_ref[1, r] = krr[:, HH:]
        vm_ref[0, r] = vmr[:, :HH]
        vm_ref[1, r] = vmr[:, HH:]


def _stage1(x_pad, mask, Wk, bk, Wq, bq, Wv, bv, bd_att, bd_msg):
    full = lambda s: pl.BlockSpec(s, lambda b: tuple(0 for _ in s))
    return pl.pallas_call(
        _stage1_body,
        grid=(NB,),
        in_specs=[
            pl.BlockSpec((BN, IN_DIM), lambda b: (b, 0)),
            pl.BlockSpec((BN, NUM_TYPES), lambda b: (b, 0)),
            full((NUM_TYPES, IN_DIM, OUT_DIM)),
            full((NUM_TYPES, OUT_DIM)),
            full((NUM_TYPES, IN_DIM, OUT_DIM)),
            full((NUM_TYPES, OUT_DIM)),
            full((NUM_TYPES, IN_DIM, OUT_DIM)),
            full((NUM_TYPES, OUT_DIM)),
            full((NUM_RELATIONS, OUT_DIM, OUT_DIM)),
            full((NUM_RELATIONS, OUT_DIM, OUT_DIM)),
        ],
        out_specs=[
            pl.BlockSpec((NC, BN, HH), lambda b: (0, b, 0)),
            pl.BlockSpec((NC, NUM_RELATIONS, BN, HH), lambda b: (0, 0, b, 0)),
            pl.BlockSpec((NC, NUM_RELATIONS, BN, HH), lambda b: (0, 0, b, 0)),
        ],
        out_shape=[
            jax.ShapeDtypeStruct((NC, NPAD, HH), jnp.float32),
            jax.ShapeDtypeStruct((NC, NUM_RELATIONS, NPAD, HH), jnp.float32),
            jax.ShapeDtypeStruct((NC, NUM_RELATIONS, NPAD, HH), jnp.float32),
        ],
    )(x_pad, mask, Wk, bk, Wq, bq, Wv, bv, bd_att, bd_msg)


def _stage3_body(agg_ref, x_ref, m_ref, wa_ref, ba_ref, skip_ref, lng_ref,
                 lnb_ref, out_ref):
    a = jnp.concatenate([agg_ref[0], agg_ref[1]], axis=1)
    x = x_ref[...]
    m = m_ref[...]
    g = jax.nn.gelu(a)
    trans = jnp.zeros((BN, OUT_DIM), jnp.float32)
    for t in range(NUM_TYPES):
        y = jnp.dot(g, wa_ref[t], preferred_element_type=jnp.float32)
        y = y + ba_ref[t][None, :]
        trans = trans + y * m[:, t][:, None]
    alpha = jnp.dot(m, jax.nn.sigmoid(skip_ref[0])[:, None],
                    preferred_element_type=jnp.float32)
    out = trans * alpha + x * (1.0 - alpha)
    mu = jnp.mean(out, axis=-1, keepdims=True)
    var = jnp.mean((out - mu) ** 2, axis=-1, keepdims=True)
    out = (out - mu) * lax.rsqrt(var + 1e-5)
    gv = jnp.dot(m, lng_ref[...], preferred_element_type=jnp.float32)
    bv = jnp.dot(m, lnb_ref[...], preferred_element_type=jnp.float32)
    out_ref[...] = out * gv + bv


def _stage3(agg, x_pad, mask, Wa, ba, skip, ln_g, ln_b):
    full = lambda s: pl.BlockSpec(s, lambda b: tuple(0 for _ in s))
    return pl.pallas_call(
        _stage3_body,
        grid=(NB,),
        in_specs=[
            pl.BlockSpec((NC, BN, HH), lambda b: (0, b, 0)),
            pl.BlockSpec((BN, IN_DIM), lambda b: (b, 0)),
            pl.BlockSpec((BN, NUM_TYPES), lambda b: (b, 0)),
            full((NUM_TYPES, OUT_DIM, OUT_DIM)),
            full((NUM_TYPES, OUT_DIM)),
            full((1, NUM_TYPES)),
            full((NUM_TYPES, OUT_DIM)),
            full((NUM_TYPES, OUT_DIM)),
        ],
        out_specs=pl.BlockSpec((BN, OUT_DIM), lambda b: (b, 0)),
        out_shape=jax.ShapeDtypeStruct((NPAD, OUT_DIM), jnp.float32),
    )(agg, x_pad, mask, Wa, ba, skip, ln_g, ln_b)


def _sc_body(qh, krt, vmt, ikr, idst, lg_o, agg_o,
             idxa, idxd, idxr, gba, gbb, lbuf, padb, mxbuf, mxrow, zbufd,
             maxsh, dpk, aggsh, sem1, sem2):
    c = lax.axis_index("c")
    sid = lax.axis_index("s")
    base_e = sid * EPT
    it = lax.iota(jnp.int32, 16)
    c4 = c * (NUM_RELATIONS * NPAD)
    c1 = c * NPAD

    def load_idx(off, with_q):
        pltpu.sync_copy(idst.at[pl.ds(off, CHUNK)], idxd)
        pltpu.sync_copy(ikr.at[pl.ds(off, CHUNK)], idxa)
        for j in range(CHUNK // 16):
            s = pl.ds(j * 16, 16)
            idxa[s] = idxa[s] + c4
            if with_q:
                idxd[s] = idxd[s] + c1

    def chunk_a(i, mx):
        off = base_e + i * CHUNK
        load_idx(off, with_q=True)
        cp1 = pltpu.async_copy(krt.at[idxa], gba, sem1)
        cp2 = pltpu.async_copy(qh.at[idxd], gbb, sem2)
        cp1.wait()
        cp2.wait()

        def group(j, mxc):
            gacc = jnp.zeros((16,), jnp.float32)
            it4 = lax.shift_right_logical(it, 2)
            for el in range(4):
                e = 4 * j + el
                parts = []
                for h in range(4):
                    p = (gbb[e, pl.ds(h * 32, 16)]
                         * gba[e, pl.ds(h * 32, 16)]
                         + gbb[e, pl.ds(h * 32 + 16, 16)]
                         * gba[e, pl.ds(h * 32 + 16, 16)])
                    p = p + _take(p, it ^ 8)
                    p = p + _take(p, it ^ 4)
                    parts.append(p)
                cc = jnp.zeros((16,), jnp.float32)
                for h in range(4):
                    cc = jnp.where(it % 4 == h, _take(parts[h], it4), cc)
                cc = cc + _take(cc, it ^ 8)
                cc = cc + _take(cc, it ^ 4)
                gacc = jnp.where(it4 == el, cc, gacc)
            lbuf[pl.ds(j * 16, 16)] = gacc
            return jnp.maximum(mxc, gacc)

        mx = lax.fori_loop(0, CHUNK // 4, group, mx)
        pltpu.sync_copy(lbuf, lg_o.at[c, pl.ds(off * 4, CHUNK * 4)])
        return mx

    mx = lax.fori_loop(0, NCH, chunk_a, jnp.full((16,), -1e30, jnp.float32))
    mxrow[0, pl.ds(0, 16)] = mx
    pltpu.sync_copy(mxrow, maxsh.at[pl.ds(sid, 1)])
    plsc.subcore_barrier()

    pltpu.sync_copy(maxsh, mxbuf)
    mall = mxbuf[0, pl.ds(0, 16)]
    for i in range(1, NS):
        mall = jnp.maximum(mall, mxbuf[i, pl.ds(0, 16)])
    mall = jnp.maximum(mall, _take(mall, (it + 4) % 16))
    mvec = jnp.maximum(mall, _take(mall, (it + 8) % 16))

    for p in range(2):
        lo = p * HN

        def zrow_a(i, carry):
            for j in range(8):
                gbb[i, pl.ds(j * 16, 16)] = jnp.zeros((16,), jnp.float32)
            return carry

        lax.fori_loop(0, CHUNK, zrow_a, 0)

        z0 = sid * ZPT
        for i in range((ZPT + CHUNK - 1) // CHUNK):
            nr = min(CHUNK, ZPT - CHUNK * i)
            pltpu.sync_copy(gbb.at[pl.ds(0, nr)],
                            aggsh.at[pl.ds(z0 + CHUNK * i, nr)])

        @pl.when(sid == 0)
        def _():
            for i in range((DPKR + CHUNK - 1) // CHUNK):
                nr = min(CHUNK, DPKR - CHUNK * i)
                pltpu.sync_copy(gbb.at[pl.ds(0, nr)],
                                dpk.at[pl.ds(CHUNK * i, nr)])

        plsc.subcore_barrier()

        def chunk_b(i, mv):
            off = base_e + i * CHUNK
            load_idx(off, with_q=False)
            cp1 = pltpu.async_copy(vmt.at[idxa], gba, sem1)
            pltpu.sync_copy(lg_o.at[c, pl.ds(off * 4, CHUNK * 4)], lbuf)
            for j in range(CHUNK // 16):
                sl = pl.ds(j * 16, 16)
                v = idxd[sl] - lo
                inb = (v >= 0) & (v < HN)
                idxr[sl] = jnp.where(inb, v, HN)
            cp1.wait()

            def group(j, carry):
                gv = lbuf[pl.ds(j * 16, 16)]
                ev = jnp.exp(gv - mv)
                for el in range(4):
                    r = jnp.where(it < 4, _take(ev, (it % 4) + 4 * el), 0.0)
                    padb[4 * j + el, pl.ds(0, 16)] = r
                return carry

            lax.fori_loop(0, CHUNK // 4, group, 0)

            for j in range(CHUNK // 16):
                sl = pl.ds(j * 16, 16)
                idxa[sl] = lax.shift_right_logical(idxr[sl], 3)

            def degrp(g, carry):
                dv = idxr[pl.ds(g * 16, 16)]
                for el in range(16):
                    e = g * 16 + el
                    for j in range(8):
                        gbb[e, pl.ds(j * 16, 16)] = jnp.zeros(
                            (16,), jnp.float32)
                    slot = (dv[el] & 7) * 16
                    gbb[e, pl.ds(slot, 16)] = padb[e, pl.ds(0, 16)]
                return carry

            lax.fori_loop(0, CHUNK // 16, degrp, 0)
            pltpu.sync_copy(gbb, dpk.at[idxa], add=True)

            def edge(e, carry):
                er = padb[e, pl.ds(0, 16)]
                for h in range(4):
                    bh = er[h]
                    s0 = pl.ds(h * 32, 16)
                    s1 = pl.ds(h * 32 + 16, 16)
                    gbb[e, s0] = gba[e, s0] * bh
                    gbb[e, s1] = gba[e, s1] * bh
                return carry

            lax.fori_loop(0, CHUNK, edge, 0)
            pltpu.sync_copy(gbb, aggsh.at[idxr], add=True)
            return mv

        mvec = lax.fori_loop(0, NCH, chunk_b, mvec)
        plsc.subcore_barrier()

        r0 = sid * DPT
        pltpu.sync_copy(dpk.at[pl.ds(sid * (DPT // 8), DPT // 8)], zbufd)
        for blk in range(DPT // CHUNK):
            pltpu.sync_copy(aggsh.at[pl.ds(r0 + CHUNK * blk, CHUNK)], gbb)

            def rowdiv(i, carry):
                nloc = CHUNK * blk + i
                drow = lax.shift_right_logical(nloc, 3)
                dcol = (nloc & 7) * 16
                dr = zbufd[drow, pl.ds(dcol, 16)]
                rec = 1.0 / (dr + 1e-16)
                for jj in range(8):
                    bh = rec[jj // 2]
                    sl = pl.ds(jj * 16, 16)
                    gbb[i, sl] = gbb[i, sl] * bh
                return carry

            lax.fori_loop(0, CHUNK, rowdiv, 0)
            pltpu.sync_copy(gbb, agg_o.at[c, pl.ds(lo + r0 + CHUNK * blk, CHUNK)])
        plsc.subcore_barrier()


def _sc_stage(qh_flat, kr_flat, vm_flat, idx_kr, idx_dst):
    mesh = plsc.VectorSubcoreMesh(core_axis_name="c", subcore_axis_name="s",
                                  num_cores=NC, num_subcores=NS)
    f = pl.kernel(
        _sc_body,
        out_type=[
            jax.ShapeDtypeStruct((NC, EPAD * 4), jnp.float32),
            jax.ShapeDtypeStruct((NC, NPAD, HH), jnp.float32),
        ],
        mesh=mesh,
        scratch_types=[
            pltpu.VMEM((CHUNK,), jnp.int32),
            pltpu.VMEM((CHUNK,), jnp.int32),
            pltpu.VMEM((CHUNK,), jnp.int32),
            pltpu.VMEM((CHUNK, HH), jnp.float32),
            pltpu.VMEM((CHUNK, HH), jnp.float32),
            pltpu.VMEM((CHUNK * 4,), jnp.float32),
            pltpu.VMEM((CHUNK, 16), jnp.float32),
            pltpu.VMEM((NS, 16), jnp.float32),
            pltpu.VMEM((1, 16), jnp.float32),
            pltpu.VMEM((DPT // 8, HH), jnp.float32),
            pltpu.VMEM_SHARED((NS, 16), jnp.float32),
            pltpu.VMEM_SHARED((DPKR, HH), jnp.float32),
            pltpu.VMEM_SHARED((HN_ACC, HH), jnp.float32),
            pltpu.SemaphoreType.DMA,
            pltpu.SemaphoreType.DMA,
        ],
    )
    return f(qh_flat, kr_flat, vm_flat, idx_kr, idx_dst)


def kernel(node_inp, node_type, edge_index, edge_type, Wk, bk, Wq, bq, Wv,
           bv, Wa, ba, relation_pri, relation_att, relation_msg, skip,
           ln_g, ln_b):
    x_pad = jnp.pad(node_inp, ((0, NPAD - N), (0, 0)))
    t_pad = jnp.pad(node_type, (0, NPAD - N))
    mask = (t_pad[:, None] == jnp.arange(NUM_TYPES)[None, :]).astype(
        jnp.float32)

    eye = jnp.eye(N_HEADS, dtype=jnp.float32)
    att_s = relation_att * (relation_pri / SQRT_DK)[:, :, None, None]
    bd_att = jnp.einsum('rhdf,hg->rhdgf', att_s, eye).reshape(
        NUM_RELATIONS, OUT_DIM, OUT_DIM)
    bd_msg = jnp.einsum('rhdf,hg->rhdgf', relation_msg, eye).reshape(
        NUM_RELATIONS, OUT_DIM, OUT_DIM)

    qh, kr, vm = _stage1(x_pad, mask, Wk, bk, Wq, bq, Wv, bv, bd_att, bd_msg)

    src = edge_index[0]
    dst = edge_index[1]
    idx_kr = jnp.pad(edge_type * NPAD + src, (0, EPAD - E))
    idx_dst = jnp.pad(dst, (0, EPAD - E), constant_values=N)

    _, agg = _sc_stage(
        qh.reshape(NC * NPAD, HH),
        kr.reshape(NC * NUM_RELATIONS * NPAD, HH),
        vm.reshape(NC * NUM_RELATIONS * NPAD, HH),
        idx_kr, idx_dst)

    out = _stage3(agg, x_pad, mask, Wa, ba, skip.reshape(1, NUM_TYPES),
                  ln_g, ln_b)
    return out[:N]


# speedup vs baseline: 2.9101x; 1.4038x over previous
import jax
import jax.numpy as jnp
from jax import lax
from jax.experimental import pallas as pl
from jax.experimental.pallas import tpu as pltpu
from jax.experimental.pallas import tpu_sc as plsc

N = 10000
E = 160000
IN_DIM = 256
OUT_DIM = 256
NUM_TYPES = 4
NUM_RELATIONS = 4
N_HEADS = 8
D_K = 32
SQRT_DK = float(32) ** 0.5

NPAD = 10240
BN = 512
NB = NPAD // BN
NC = 2
NS = 16
CHUNK = 64
NCH = 160
EPT = NCH * CHUNK
EPAD = NS * EPT
HH = 128
HN = NPAD // 2
HN_ACC = HN + 128
ZPT = HN_ACC // NS
DPT = HN // NS
DPKR = 648

def _take(v, idx):
    return v.at[idx].get(mode="promise_in_bounds")


def _stage1_body(x_ref, m_ref, wk_ref, bk_ref, wq_ref, bq_ref, wv_ref,
                 bv_ref, bda_ref, bdm_ref, qh_ref, kr_ref, vm_ref):
    x = x_ref[...]
    m = m_ref[...]

    def typed(w_ref, b_ref):
        acc = jnp.zeros((BN, OUT_DIM), jnp.float32)
        for t in range(NUM_TYPES):
            y = jnp.dot(x, w_ref[t], preferred_element_type=jnp.float32)
            y = y + b_ref[t][None, :]
            acc = acc + y * m[:, t][:, None]
        return acc

    k = typed(wk_ref, bk_ref)
    q = typed(wq_ref, bq_ref)
    v = typed(wv_ref, bv_ref)
    qh_ref[0] = q[:, :HH]
    qh_ref[1] = q[:, HH:]
    for r in range(NUM_RELATIONS):
        krr = jnp.dot(k, bda_ref[r], preferred_element_type=jnp.float32)
        vmr = jnp.dot(v, bdm_ref[r], preferred_element_type=jnp.float32)
        kr_ref[0, r] = krr[:, :HH]
        kr_ref[1, r] = krr[:, HH:]
        vm_ref[0, r] = vmr[:, :HH]
        vm_ref[1, r] = vmr[:, HH:]


def _stage1(x_pad, mask, Wk, bk, Wq, bq, Wv, bv, bd_att, bd_msg):
    full = lambda s: pl.BlockSpec(s, lambda b: tuple(0 for _ in s))
    return pl.pallas_call(
        _stage1_body,
        grid=(NB,),
        in_specs=[
            pl.BlockSpec((BN, IN_DIM), lambda b: (b, 0)),
            pl.BlockSpec((BN, NUM_TYPES), lambda b: (b, 0)),
            full((NUM_TYPES, IN_DIM, OUT_DIM)),
            full((NUM_TYPES, OUT_DIM)),
            full((NUM_TYPES, IN_DIM, OUT_DIM)),
            full((NUM_TYPES, OUT_DIM)),
            full((NUM_TYPES, IN_DIM, OUT_DIM)),
            full((NUM_TYPES, OUT_DIM)),
            full((NUM_RELATIONS, OUT_DIM, OUT_DIM)),
            full((NUM_RELATIONS, OUT_DIM, OUT_DIM)),
        ],
        out_specs=[
            pl.BlockSpec((NC, BN, HH), lambda b: (0, b, 0)),
            pl.BlockSpec((NC, NUM_RELATIONS, BN, HH), lambda b: (0, 0, b, 0)),
            pl.BlockSpec((NC, NUM_RELATIONS, BN, HH), lambda b: (0, 0, b, 0)),
        ],
        out_shape=[
            jax.ShapeDtypeStruct((NC, NPAD, HH), jnp.float32),
            jax.ShapeDtypeStruct((NC, NUM_RELATIONS, NPAD, HH), jnp.float32),
            jax.ShapeDtypeStruct((NC, NUM_RELATIONS, NPAD, HH), jnp.float32),
        ],
    )(x_pad, mask, Wk, bk, Wq, bq, Wv, bv, bd_att, bd_msg)


def _stage3_body(agg_ref, x_ref, m_ref, wa_ref, ba_ref, skip_ref, lng_ref,
                 lnb_ref, out_ref):
    a = jnp.concatenate([agg_ref[0], agg_ref[1]], axis=1)
    x = x_ref[...]
    m = m_ref[...]
    g = jax.nn.gelu(a)
    trans = jnp.zeros((BN, OUT_DIM), jnp.float32)
    for t in range(NUM_TYPES):
        y = jnp.dot(g, wa_ref[t], preferred_element_type=jnp.float32)
        y = y + ba_ref[t][None, :]
        trans = trans + y * m[:, t][:, None]
    alpha = jnp.dot(m, jax.nn.sigmoid(skip_ref[0])[:, None],
                    preferred_element_type=jnp.float32)
    out = trans * alpha + x * (1.0 - alpha)
    mu = jnp.mean(out, axis=-1, keepdims=True)
    var = jnp.mean((out - mu) ** 2, axis=-1, keepdims=True)
    out = (out - mu) * lax.rsqrt(var + 1e-5)
    gv = jnp.dot(m, lng_ref[...], preferred_element_type=jnp.float32)
    bv = jnp.dot(m, lnb_ref[...], preferred_element_type=jnp.float32)
    out_ref[...] = out * gv + bv


def _stage3(agg, x_pad, mask, Wa, ba, skip, ln_g, ln_b):
    full = lambda s: pl.BlockSpec(s, lambda b: tuple(0 for _ in s))
    return pl.pallas_call(
        _stage3_body,
        grid=(NB,),
        in_specs=[
            pl.BlockSpec((NC, BN, HH), lambda b: (0, b, 0)),
            pl.BlockSpec((BN, IN_DIM), lambda b: (b, 0)),
            pl.BlockSpec((BN, NUM_TYPES), lambda b: (b, 0)),
            full((NUM_TYPES, OUT_DIM, OUT_DIM)),
            full((NUM_TYPES, OUT_DIM)),
            full((1, NUM_TYPES)),
            full((NUM_TYPES, OUT_DIM)),
            full((NUM_TYPES, OUT_DIM)),
        ],
        out_specs=pl.BlockSpec((BN, OUT_DIM), lambda b: (b, 0)),
        out_shape=jax.ShapeDtypeStruct((NPAD, OUT_DIM), jnp.float32),
    )(agg, x_pad, mask, Wa, ba, skip, ln_g, ln_b)


SUP = 8
NSUP = NCH // SUP


def _sc_body(qh, krt, vmt, ikr, idst, lg_o, agg_o,
             ikrb, idstb, idxr, idxden, gba, gbb, gbc, lbuf, padb,
             mxbuf, mxrow, zbufd, maxsh, dpk, aggsh, sem1, sem2, sem3, sem4):
    c = lax.axis_index("c")
    sid = lax.axis_index("s")
    base_e = sid * EPT
    it = lax.iota(jnp.int32, 16)
    c4 = c * (NUM_RELATIONS * NPAD)
    c1 = c * NPAD
    SC = SUP * CHUNK

    def sup_a(u, mx):
        offs = base_e + u * SC
        pltpu.sync_copy(idst.at[pl.ds(offs, SC)], idstb)
        pltpu.sync_copy(ikr.at[pl.ds(offs, SC)], ikrb)
        for j in range(SC // 16):
            sl = pl.ds(j * 16, 16)
            ikrb[sl] = ikrb[sl] + c4
            idstb[sl] = idstb[sl] + c1
        def chunk_a(k, mxk):
            cp1 = pltpu.async_copy(
                krt.at[ikrb.at[pl.ds(k * CHUNK, CHUNK)]], gba, sem1)
            cp2 = pltpu.async_copy(
                qh.at[idstb.at[pl.ds(k * CHUNK, CHUNK)]], gbb, sem2)
            cp1.wait()
            cp2.wait()

            def group(j, mxc):
                gacc = jnp.zeros((16,), jnp.float32)
                it4 = lax.shift_right_logical(it, 2)
                for el in range(4):
                    e = 4 * j + el
                    parts = []
                    for h in range(4):
                        p = (gbb[e, pl.ds(h * 32, 16)]
                             * gba[e, pl.ds(h * 32, 16)]
                             + gbb[e, pl.ds(h * 32 + 16, 16)]
                             * gba[e, pl.ds(h * 32 + 16, 16)])
                        p = p + _take(p, it ^ 8)
                        p = p + _take(p, it ^ 4)
                        parts.append(p)
                    cc = jnp.zeros((16,), jnp.float32)
                    for h in range(4):
                        cc = jnp.where(it % 4 == h, _take(parts[h], it4), cc)
                    cc = cc + _take(cc, it ^ 8)
                    cc = cc + _take(cc, it ^ 4)
                    gacc = jnp.where(it4 == el, cc, gacc)
                lbuf[pl.ds(k * (CHUNK * 4) + j * 16, 16)] = gacc
                return jnp.maximum(mxc, gacc)

            return lax.fori_loop(0, CHUNK // 4, group, mxk)

        mx = lax.fori_loop(0, SUP, chunk_a, mx)
        pltpu.sync_copy(lbuf, lg_o.at[c, pl.ds(offs * 4, SC * 4)])
        return mx

    mx = lax.fori_loop(0, NSUP, sup_a, jnp.full((16,), -1e30, jnp.float32))
    mxrow[0, pl.ds(0, 16)] = mx
    pltpu.sync_copy(mxrow, maxsh.at[pl.ds(sid, 1)])
    plsc.subcore_barrier()

    pltpu.sync_copy(maxsh, mxbuf)
    mall = mxbuf[0, pl.ds(0, 16)]
    for i in range(1, NS):
        mall = jnp.maximum(mall, mxbuf[i, pl.ds(0, 16)])
    mall = jnp.maximum(mall, _take(mall, (it + 4) % 16))
    mvec = jnp.maximum(mall, _take(mall, (it + 8) % 16))

    for p in range(2):
        lo = p * HN

        def zrow_a(i, carry):
            for j in range(8):
                gbb[i, pl.ds(j * 16, 16)] = jnp.zeros((16,), jnp.float32)
            return carry

        lax.fori_loop(0, CHUNK, zrow_a, 0)

        z0 = sid * ZPT
        for i in range((ZPT + CHUNK - 1) // CHUNK):
            nr = min(CHUNK, ZPT - CHUNK * i)
            pltpu.sync_copy(gbb.at[pl.ds(0, nr)],
                            aggsh.at[pl.ds(z0 + CHUNK * i, nr)])

        @pl.when(sid == 0)
        def _():
            for i in range((DPKR + CHUNK - 1) // CHUNK):
                nr = min(CHUNK, DPKR - CHUNK * i)
                pltpu.sync_copy(gbb.at[pl.ds(0, nr)],
                                dpk.at[pl.ds(CHUNK * i, nr)])

        plsc.subcore_barrier()

        def sup_b(u, mv):
            offs = base_e + u * SC
            pltpu.sync_copy(idst.at[pl.ds(offs, SC)], idstb)
            pltpu.sync_copy(ikr.at[pl.ds(offs, SC)], ikrb)
            for j in range(SC // 16):
                sl = pl.ds(j * 16, 16)
                ikrb[sl] = ikrb[sl] + c4
            pltpu.sync_copy(lg_o.at[c, pl.ds(offs * 4, SC * 4)], lbuf)
            def chunk_b(k, mvk):
                cpv = pltpu.async_copy(
                    vmt.at[ikrb.at[pl.ds(k * CHUNK, CHUNK)]], gba, sem1)

                @pl.when((u > 0) | (k > 0))
                def _():
                    pltpu.make_async_copy(gbc, dpk.at[idxden], sem3).wait()
                    pltpu.make_async_copy(gbb, aggsh.at[idxr], sem4).wait()
                for j in range(CHUNK // 16):
                    sl = pl.ds(j * 16, 16)
                    v = idstb[pl.ds(k * CHUNK + j * 16, 16)] - lo
                    inb = (v >= 0) & (v < HN)
                    idxr[sl] = jnp.where(inb, v, HN)
                    idxden[sl] = lax.shift_right_logical(idxr[sl], 3)

                def group(j, carry):
                    gv = lbuf[pl.ds(k * (CHUNK * 4) + j * 16, 16)]
                    ev = jnp.exp(gv - mv)
                    for el in range(4):
                        r = jnp.where(it < 4,
                                      _take(ev, (it % 4) + 4 * el), 0.0)
                        padb[4 * j + el, pl.ds(0, 16)] = r
                    return carry

                lax.fori_loop(0, CHUNK // 4, group, 0)

                def degrp(g, carry):
                    dv = idxr[pl.ds(g * 16, 16)]
                    for el in range(16):
                        e = g * 16 + el
                        for j in range(8):
                            gbc[e, pl.ds(j * 16, 16)] = jnp.zeros(
                                (16,), jnp.float32)
                        slot = (dv[el] & 7) * 16
                        gbc[e, pl.ds(slot, 16)] = padb[e, pl.ds(0, 16)]
                    return carry

                lax.fori_loop(0, CHUNK // 16, degrp, 0)
                cpv.wait()
                pltpu.async_copy(gbc, dpk.at[idxden], sem3, add=True)

                def edge(e, carry):
                    er = padb[e, pl.ds(0, 16)]
                    for h in range(4):
                        bh = er[h]
                        s0 = pl.ds(h * 32, 16)
                        s1 = pl.ds(h * 32 + 16, 16)
                        gbb[e, s0] = gba[e, s0] * bh
                        gbb[e, s1] = gba[e, s1] * bh
                    return carry

                lax.fori_loop(0, CHUNK, edge, 0)
                pltpu.async_copy(gbb, aggsh.at[idxr], sem4, add=True)
                return mvk

            return lax.fori_loop(0, SUP, chunk_b, mv)

        mvec = lax.fori_loop(0, NSUP, sup_b, mvec)
        pltpu.make_async_copy(gbc, dpk.at[idxden], sem3).wait()
        pltpu.make_async_copy(gbb, aggsh.at[idxr], sem4).wait()
        plsc.subcore_barrier()

        r0 = sid * DPT
        for blk in range(DPT // CHUNK):
            pltpu.sync_copy(
                dpk.at[pl.ds(sid * (DPT // 8) + 8 * blk, CHUNK // 8)], zbufd)
            pltpu.sync_copy(aggsh.at[pl.ds(r0 + CHUNK * blk, CHUNK)], gbb)

            def rowdiv(i, carry):
                drow = lax.shift_right_logical(i, 3)
                dcol = (i & 7) * 16
                dr = zbufd[drow, pl.ds(dcol, 16)]
                rec = 1.0 / (dr + 1e-16)
                for jj in range(8):
                    bh = rec[jj // 2]
                    sl = pl.ds(jj * 16, 16)
                    gbb[i, sl] = gbb[i, sl] * bh
                return carry

            lax.fori_loop(0, CHUNK, rowdiv, 0)
            pltpu.sync_copy(gbb,
                            agg_o.at[c, pl.ds(lo + r0 + CHUNK * blk, CHUNK)])
        plsc.subcore_barrier()


def _sc_stage(qh_flat, kr_flat, vm_flat, idx_kr, idx_dst):
    mesh = plsc.VectorSubcoreMesh(core_axis_name="c", subcore_axis_name="s",
                                  num_cores=NC, num_subcores=NS)
    f = pl.kernel(
        _sc_body,
        out_type=[
            jax.ShapeDtypeStruct((NC, EPAD * 4), jnp.float32),
            jax.ShapeDtypeStruct((NC, NPAD, HH), jnp.float32),
        ],
        mesh=mesh,
        scratch_types=[
            pltpu.VMEM((SUP * CHUNK,), jnp.int32),
            pltpu.VMEM((SUP * CHUNK,), jnp.int32),
            pltpu.VMEM((CHUNK,), jnp.int32),
            pltpu.VMEM((CHUNK,), jnp.int32),
            pltpu.VMEM((CHUNK, HH), jnp.float32),
            pltpu.VMEM((CHUNK, HH), jnp.float32),
            pltpu.VMEM((CHUNK, HH), jnp.float32),
            pltpu.VMEM((SUP * CHUNK * 4,), jnp.float32),
            pltpu.VMEM((CHUNK, 16), jnp.float32),
            pltpu.VMEM((NS, 16), jnp.float32),
            pltpu.VMEM((1, 16), jnp.float32),
            pltpu.VMEM((CHUNK // 8, HH), jnp.float32),
            pltpu.VMEM_SHARED((NS, 16), jnp.float32),
            pltpu.VMEM_SHARED((DPKR, HH), jnp.float32),
            pltpu.VMEM_SHARED((HN_ACC, HH), jnp.float32),
            pltpu.SemaphoreType.DMA,
            pltpu.SemaphoreType.DMA,
            pltpu.SemaphoreType.DMA,
            pltpu.SemaphoreType.DMA,
        ],
    )
    return f(qh_flat, kr_flat, vm_flat, idx_kr, idx_dst)


def kernel(node_inp, node_type, edge_index, edge_type, Wk, bk, Wq, bq, Wv,
           bv, Wa, ba, relation_pri, relation_att, relation_msg, skip,
           ln_g, ln_b):
    x_pad = jnp.pad(node_inp, ((0, NPAD - N), (0, 0)))
    t_pad = jnp.pad(node_type, (0, NPAD - N))
    mask = (t_pad[:, None] == jnp.arange(NUM_TYPES)[None, :]).astype(
        jnp.float32)

    eye = jnp.eye(N_HEADS, dtype=jnp.float32)
    att_s = relation_att * (relation_pri / SQRT_DK)[:, :, None, None]
    bd_att = jnp.einsum('rhdf,hg->rhdgf', att_s, eye).reshape(
        NUM_RELATIONS, OUT_DIM, OUT_DIM)
    bd_msg = jnp.einsum('rhdf,hg->rhdgf', relation_msg, eye).reshape(
        NUM_RELATIONS, OUT_DIM, OUT_DIM)

    qh, kr, vm = _stage1(x_pad, mask, Wk, bk, Wq, bq, Wv, bv, bd_att, bd_msg)

    src = edge_index[0]
    dst = edge_index[1]
    idx_kr = jnp.pad(edge_type * NPAD + src, (0, EPAD - E))
    idx_dst = jnp.pad(dst, (0, EPAD - E), constant_values=N)

    _, agg = _sc_stage(
        qh.reshape(NC * NPAD, HH),
        kr.reshape(NC * NUM_RELATIONS * NPAD, HH),
        vm.reshape(NC * NUM_RELATIONS * NPAD, HH),
        idx_kr, idx_dst)

    out = _stage3(agg, x_pad, mask, Wa, ba, skip.reshape(1, NUM_TYPES),
                  ln_g, ln_b)
    return out[:N]
